# baseline (speedup 1.0000x reference)
import functools

import jax
import jax.numpy as jnp
from jax import lax
from jax.experimental import pallas as pl
from jax.experimental.pallas import tpu as pltpu
from jax.experimental.pallas import tpu_sc as plsc

N_ENT = 100000
N_REL = 16
D_IN = 128
D_REL = 64
M = 16384
REG_KG = 0.01

NSPLIT = 2
H = M // NSPLIT

NC, NS = 2, 16
NW = NC * NS
CHUNK = 128
PER_ARR = H // NW
N_CHUNK = PER_ARR // CHUNK


def _sc_gather_body(part, h_hbm, p_hbm, n_hbm, table_hbm, out_hbm,
                    idx_v, buf0, buf1, sem0, sem1):
    wid = lax.axis_index("s") * NC + lax.axis_index("c")
    src_base = part * H + wid * PER_ARR
    for t, a_hbm in enumerate((h_hbm, p_hbm, n_hbm)):
        pltpu.sync_copy(a_hbm.at[pl.ds(src_base, PER_ARR)],
                        idx_v.at[pl.ds(t * PER_ARR, PER_ARR)])
    bufs = (buf0, buf1)
    sems = (sem0, sem1)
    total = 3 * N_CHUNK
    handles = [None] * total
    handles[0] = pltpu.async_copy(
        table_hbm.at[idx_v.at[pl.ds(0, CHUNK)]], buf0, sem0)
    for c in range(total):
        if c + 1 < total:
            handles[c + 1] = pltpu.async_copy(
                table_hbm.at[idx_v.at[pl.ds((c + 1) * CHUNK, CHUNK)]],
                bufs[(c + 1) % 2], sems[(c + 1) % 2])
        handles[c].wait()
        t, j = divmod(c, N_CHUNK)
        dst = t * H + wid * PER_ARR + j * CHUNK
        pltpu.sync_copy(bufs[c % 2], out_hbm.at[pl.ds(dst, CHUNK)])


@functools.cache
def _sc_gather(part):
    return pl.kernel(
        functools.partial(_sc_gather_body, part),
        mesh=plsc.VectorSubcoreMesh(core_axis_name="c", subcore_axis_name="s"),
        out_type=jax.ShapeDtypeStruct((3 * H, D_IN), jnp.float32),
        scratch_types=[
            pltpu.VMEM((3 * PER_ARR,), jnp.int32),
            pltpu.VMEM((CHUNK, D_IN), jnp.float32),
            pltpu.VMEM((CHUNK, D_IN), jnp.float32),
            pltpu.SemaphoreType.DMA,
            pltpu.SemaphoreType.DMA,
        ],
    )


BLK = 1024
NCOL = N_REL * D_REL


def _normalize(x):
    n2 = jnp.sum(x * x, axis=1, keepdims=True)
    return x / jnp.maximum(jnp.sqrt(n2), 1e-12)


def _tc_body(gath_ref, r_ref, wall_ref, rel_ref, g_ref, out_ref):
    i = pl.program_id(0)
    r_col = r_ref[...]
    wall = wall_ref[...].astype(jnp.bfloat16)
    g = g_ref[...]

    col_rel = lax.broadcasted_iota(jnp.int32, (BLK, NCOL), 1) // D_REL
    colmask = col_rel == r_col

    def project(x):
        y = lax.dot_general(x.astype(jnp.bfloat16), wall,
                            (((1,), (0,)), ((), ())),
                            preferred_element_type=jnp.float32)
        y = jnp.where(colmask, y, 0.0).astype(jnp.bfloat16)
        return lax.dot_general(y, g, (((1,), (0,)), ((), ())),
                               preferred_element_type=jnp.float32)

    h_vec = _normalize(project(gath_ref[0]))
    pos_t_vec = _normalize(project(gath_ref[1]))
    neg_t_vec = _normalize(project(gath_ref[2]))

    onehot = (r_col == lax.broadcasted_iota(jnp.int32, (BLK, N_REL), 1)
              ).astype(jnp.float32)
    r_vec = _normalize(lax.dot_general(onehot, rel_ref[...],
                                       (((1,), (0,)), ((), ())),
                                       preferred_element_type=jnp.float32))

    d_pos = h_vec + r_vec - pos_t_vec
    d_neg = h_vec + r_vec - neg_t_vec
    pos_score = jnp.sum(d_pos * d_pos, axis=1, keepdims=True)
    neg_score = jnp.sum(d_neg * d_neg, axis=1, keepdims=True)
    z = neg_score - pos_score
    li = jnp.maximum(-z, 0.0) + jnp.log(1.0 + jnp.exp(-jnp.abs(z)))

    reg = 0.5 * (jnp.sum(h_vec * h_vec) + jnp.sum(r_vec * r_vec)
                 + jnp.sum(pos_t_vec * pos_t_vec)
                 + jnp.sum(neg_t_vec * neg_t_vec))
    partial = ((jnp.sum(li) + REG_KG * reg) * (1.0 / M)).reshape(1, 1)

    acc = jnp.where(i == 0, partial, out_ref[...] + partial)
    out_ref[...] = acc


@functools.cache
def _tc_compute(part):
    grid = H // BLK
    blk_off = part * grid
    return pl.pallas_call(
        _tc_body,
        grid=(grid,),
        in_specs=[
            pl.BlockSpec((3, BLK, D_IN), lambda i: (0, i, 0)),
            pl.BlockSpec((BLK, 1), lambda i: (blk_off + i, 0)),
            pl.BlockSpec((D_IN, NCOL), lambda i: (0, 0)),
            pl.BlockSpec((N_REL, D_REL), lambda i: (0, 0)),
            pl.BlockSpec((NCOL, D_REL), lambda i: (0, 0)),
        ],
        out_specs=pl.BlockSpec((1, 1), lambda i: (0, 0)),
        out_shape=jax.ShapeDtypeStruct((1, 1), jnp.float32),
    )


def kernel(entity_table, relation_table, W_R, h, r, pos_t, neg_t):
    h = h.astype(jnp.int32)
    r = r.astype(jnp.int32)
    pos_t = pos_t.astype(jnp.int32)
    neg_t = neg_t.astype(jnp.int32)
    wall = jnp.transpose(W_R, (1, 0, 2)).reshape(D_IN, NCOL)
    g = (jnp.arange(NCOL, dtype=jnp.int32)[:, None] % D_REL
         == jnp.arange(D_REL, dtype=jnp.int32)[None, :]).astype(jnp.bfloat16)
    r2d = r.reshape(M, 1)

    parts = [_sc_gather(p)(h, pos_t, neg_t, entity_table).reshape(3, H, D_IN)
             for p in range(NSPLIT)]
    out = None
    for p in range(NSPLIT):
        o = _tc_compute(p)(parts[p], r2d, wall, relation_table, g)
        out = o if out is None else out + o
    return out[0, 0]
